# Initial kernel scaffold; baseline (speedup 1.0000x reference)
import jax
import jax.numpy as jnp
from jax.experimental import pallas as pl


def kernel(a, b, sign, emb_table, ctx_table):
    raise NotImplementedError("write your pallas kernel here")


# speedup vs baseline: 1.1588x; 1.1588x over previous
import functools

import jax
import jax.numpy as jnp
from jax import lax
from jax.experimental import pallas as pl
from jax.experimental.pallas import tpu as pltpu
from jax.experimental.pallas import tpu_sc as plsc

NODE_SIZE = 100000
EMBED_SIZE = 128
BATCH = 16384

L = 16
NW = 32
B_PER_W = BATCH // NW
CHUNK = 128
NCHUNK = B_PER_W // CHUNK

_LOG1P_COEF = (
    1.446112683e-03, -1.027147447e-02, 3.423174471e-02, -7.301764925e-02,
    1.166124657e-01, -1.571737904e-01, 1.976391457e-01, -2.496172750e-01,
    3.332960370e-01, -4.999980978e-01, 9.999999616e-01, 0.0,
)


def _log1p_poly(u):
    acc = jnp.full((L,), _LOG1P_COEF[0], dtype=jnp.float32)
    for c in _LOG1P_COEF[1:]:
        acc = acc * u + c
    return acc


def _sc_kernel(a_hbm, b_hbm, sign_hbm, emb_hbm, ctx_hbm, out_hbm,
               idx_a, idx_b, sign_v, arows, brows, dots,
               sem_a, sem_b, sem_s):
    wid = lax.axis_index("s") * 2 + lax.axis_index("c")
    row0 = wid * NCHUNK
    base = wid * B_PER_W

    pltpu.sync_copy(a_hbm.at[pl.ds(row0, NCHUNK)], idx_a)
    pltpu.sync_copy(b_hbm.at[pl.ds(row0, NCHUNK)], idx_b)
    pltpu.sync_copy(sign_hbm.at[pl.ds(base, B_PER_W)], sign_v)

    lane = lax.iota(jnp.int32, L)
    mask_last = lane == (L - 1)

    for c in range(NCHUNK):
        cp_a = pltpu.async_copy(emb_hbm.at[idx_a.at[c]], arows, sem_a)
        cp_b = pltpu.async_copy(ctx_hbm.at[idx_b.at[c]], brows, sem_b)
        cp_a.wait()
        cp_b.wait()

        def dot_body(r, _, c=c):
            acc = (arows[r, pl.ds(0, L)] * brows[r, pl.ds(0, L)])
            for j in range(1, EMBED_SIZE // L):
                acc = acc + arows[r, pl.ds(j * L, L)] * brows[r, pl.ds(j * L, L)]
            cum = jnp.cumsum(acc)
            plsc.store_scatter(dots, [jnp.full((L,), c * CHUNK + r, jnp.int32)],
                               cum, mask=mask_last)
            return _

        lax.fori_loop(0, CHUNK, dot_body, None)

    def loss_body(i, _):
        off = pl.multiple_of(i * L, L)
        t = sign_v[pl.ds(off, L)] * dots[pl.ds(off, L)]
        u = jnp.exp(-jnp.abs(t))
        loss = jnp.maximum(-t, 0.0) + _log1p_poly(u)
        dots[pl.ds(off, L)] = loss
        return _

    lax.fori_loop(0, B_PER_W // L, loss_body, None)

    pltpu.sync_copy(dots, out_hbm.at[pl.ds(base, B_PER_W)])


@jax.jit
def _run(a2d, b2d, sign, emb_table, ctx_table):
    mesh = plsc.VectorSubcoreMesh(core_axis_name="c", subcore_axis_name="s")
    f = pl.kernel(
        _sc_kernel,
        mesh=mesh,
        compiler_params=pltpu.CompilerParams(needs_layout_passes=False),
        out_type=jax.ShapeDtypeStruct((BATCH,), jnp.float32),
        scratch_types=[
            pltpu.VMEM((NCHUNK, CHUNK), jnp.int32),
            pltpu.VMEM((NCHUNK, CHUNK), jnp.int32),
            pltpu.VMEM((B_PER_W,), jnp.float32),
            pltpu.VMEM((CHUNK, EMBED_SIZE), jnp.float32),
            pltpu.VMEM((CHUNK, EMBED_SIZE), jnp.float32),
            pltpu.VMEM((B_PER_W,), jnp.float32),
            pltpu.SemaphoreType.DMA,
            pltpu.SemaphoreType.DMA,
            pltpu.SemaphoreType.DMA,
        ],
    )
    return f(a2d, b2d, sign, emb_table, ctx_table)


def kernel(a, b, sign, emb_table, ctx_table):
    a2d = a.reshape(BATCH // CHUNK, CHUNK)
    b2d = b.reshape(BATCH // CHUNK, CHUNK)
    return _run(a2d, b2d, sign, emb_table, ctx_table)


# speedup vs baseline: 1.3292x; 1.1471x over previous
import functools

import jax
import jax.numpy as jnp
from jax import lax
from jax.experimental import pallas as pl
from jax.experimental.pallas import tpu as pltpu
from jax.experimental.pallas import tpu_sc as plsc

NODE_SIZE = 100000
EMBED_SIZE = 128
BATCH = 16384

L = 16
NW = 32
B_PER_W = BATCH // NW
CHUNK = 128
NCHUNK = B_PER_W // CHUNK
UNROLL = 4

_LOG1P_COEF = (
    1.446112683e-03, -1.027147447e-02, 3.423174471e-02, -7.301764925e-02,
    1.166124657e-01, -1.571737904e-01, 1.976391457e-01, -2.496172750e-01,
    3.332960370e-01, -4.999980978e-01, 9.999999616e-01, 0.0,
)


def _log1p_poly(u):
    acc = jnp.full((L,), _LOG1P_COEF[0], dtype=jnp.float32)
    for c in _LOG1P_COEF[1:]:
        acc = acc * u + c
    return acc


def _sc_kernel(a_hbm, b_hbm, sign_hbm, emb_hbm, ctx_hbm, out_hbm,
               idx_a, idx_b, sign_v, arows0, brows0, arows1, brows1, dots,
               sem_a0, sem_b0, sem_a1, sem_b1):
    wid = lax.axis_index("s") * 2 + lax.axis_index("c")
    row0 = wid * NCHUNK
    base = wid * B_PER_W

    pltpu.sync_copy(a_hbm.at[pl.ds(row0, NCHUNK)], idx_a)
    pltpu.sync_copy(b_hbm.at[pl.ds(row0, NCHUNK)], idx_b)
    pltpu.sync_copy(sign_hbm.at[pl.ds(base, B_PER_W)], sign_v)

    lane = lax.iota(jnp.int32, L)
    mask_last = lane == (L - 1)

    bufs = ((arows0, brows0, sem_a0, sem_b0), (arows1, brows1, sem_a1, sem_b1))

    def start(c):
        arows, brows, sem_a, sem_b = bufs[c % 2]
        cp_a = pltpu.async_copy(emb_hbm.at[idx_a.at[c]], arows, sem_a)
        cp_b = pltpu.async_copy(ctx_hbm.at[idx_b.at[c]], brows, sem_b)
        return cp_a, cp_b

    inflight = start(0)
    for c in range(NCHUNK):
        arows, brows, _, _ = bufs[c % 2]
        cp_a, cp_b = inflight
        cp_a.wait()
        cp_b.wait()
        if c + 1 < NCHUNK:
            inflight = start(c + 1)

        def dot_body(r2, _, c=c, arows=arows, brows=brows):
            r = pl.multiple_of(r2 * UNROLL, UNROLL)
            for k in range(UNROLL):
                acc0 = (arows[r + k, pl.ds(0, L)] * brows[r + k, pl.ds(0, L)])
                acc1 = (arows[r + k, pl.ds(L, L)] * brows[r + k, pl.ds(L, L)])
                for j in range(2, EMBED_SIZE // L, 2):
                    acc0 = acc0 + arows[r + k, pl.ds(j * L, L)] * brows[r + k, pl.ds(j * L, L)]
                    acc1 = acc1 + arows[r + k, pl.ds((j + 1) * L, L)] * brows[r + k, pl.ds((j + 1) * L, L)]
                cum = jnp.cumsum(acc0 + acc1)
                plsc.store_scatter(
                    dots, [jnp.full((L,), c * CHUNK + r + k, jnp.int32)],
                    cum, mask=mask_last)
            return _

        lax.fori_loop(0, CHUNK // UNROLL, dot_body, None)

    def loss_body(i2, _):
        for k in range(UNROLL):
            off = pl.multiple_of((i2 * UNROLL + k) * L, L)
            t = sign_v[pl.ds(off, L)] * dots[pl.ds(off, L)]
            u = jnp.exp(-jnp.abs(t))
            loss = jnp.maximum(-t, 0.0) + _log1p_poly(u)
            dots[pl.ds(off, L)] = loss
        return _

    lax.fori_loop(0, B_PER_W // L // UNROLL, loss_body, None)

    pltpu.sync_copy(dots, out_hbm.at[pl.ds(base, B_PER_W)])


@jax.jit
def _run(a2d, b2d, sign, emb_table, ctx_table):
    mesh = plsc.VectorSubcoreMesh(core_axis_name="c", subcore_axis_name="s")
    f = pl.kernel(
        _sc_kernel,
        mesh=mesh,
        compiler_params=pltpu.CompilerParams(needs_layout_passes=False),
        out_type=jax.ShapeDtypeStruct((BATCH,), jnp.float32),
        scratch_types=[
            pltpu.VMEM((NCHUNK, CHUNK), jnp.int32),
            pltpu.VMEM((NCHUNK, CHUNK), jnp.int32),
            pltpu.VMEM((B_PER_W,), jnp.float32),
            pltpu.VMEM((CHUNK, EMBED_SIZE), jnp.float32),
            pltpu.VMEM((CHUNK, EMBED_SIZE), jnp.float32),
            pltpu.VMEM((CHUNK, EMBED_SIZE), jnp.float32),
            pltpu.VMEM((CHUNK, EMBED_SIZE), jnp.float32),
            pltpu.VMEM((B_PER_W,), jnp.float32),
            pltpu.SemaphoreType.DMA,
            pltpu.SemaphoreType.DMA,
            pltpu.SemaphoreType.DMA,
            pltpu.SemaphoreType.DMA,
        ],
    )
    return f(a2d, b2d, sign, emb_table, ctx_table)


def kernel(a, b, sign, emb_table, ctx_table):
    a2d = a.reshape(BATCH // CHUNK, CHUNK)
    b2d = b.reshape(BATCH // CHUNK, CHUNK)
    return _run(a2d, b2d, sign, emb_table, ctx_table)


# speedup vs baseline: 1.4807x; 1.1140x over previous
import functools

import jax
import jax.numpy as jnp
from jax import lax
from jax.experimental import pallas as pl
from jax.experimental.pallas import tpu as pltpu
from jax.experimental.pallas import tpu_sc as plsc

NODE_SIZE = 100000
EMBED_SIZE = 128
BATCH = 16384

L = 16
NW = 32
B_PER_W = BATCH // NW
CHUNK = 128
NCHUNK = B_PER_W // CHUNK
UNROLL = 4

_LOG1P_COEF = (
    1.446112683e-03, -1.027147447e-02, 3.423174471e-02, -7.301764925e-02,
    1.166124657e-01, -1.571737904e-01, 1.976391457e-01, -2.496172750e-01,
    3.332960370e-01, -4.999980978e-01, 9.999999616e-01, 0.0,
)


def _log1p_poly(u):
    acc = jnp.full((L,), _LOG1P_COEF[0], dtype=jnp.float32)
    for c in _LOG1P_COEF[1:]:
        acc = acc * u + c
    return acc


def _sc_kernel(a_hbm, b_hbm, sign_hbm, emb_hbm, ctx_hbm, out_hbm,
               idx_a, idx_b, sign_v, arows0, brows0, arows1, brows1, dots,
               sem_a0, sem_b0, sem_a1, sem_b1):
    wid = lax.axis_index("s") * 2 + lax.axis_index("c")
    row0 = wid * NCHUNK
    base = wid * B_PER_W

    pltpu.sync_copy(a_hbm.at[pl.ds(row0, NCHUNK)], idx_a)
    pltpu.sync_copy(b_hbm.at[pl.ds(row0, NCHUNK)], idx_b)
    pltpu.sync_copy(sign_hbm.at[pl.ds(base, B_PER_W)], sign_v)

    lane = lax.iota(jnp.int32, L)
    mask_last = lane == (L - 1)

    bufs = ((arows0, brows0, sem_a0, sem_b0), (arows1, brows1, sem_a1, sem_b1))

    def start(c):
        arows, brows, sem_a, sem_b = bufs[c % 2]
        cp_a = pltpu.async_copy(emb_hbm.at[idx_a.at[c]], arows, sem_a)
        cp_b = pltpu.async_copy(ctx_hbm.at[idx_b.at[c]], brows, sem_b)
        return cp_a, cp_b

    inflight = start(0)
    for c in range(NCHUNK):
        arows, brows, _, _ = bufs[c % 2]
        cp_a, cp_b = inflight
        cp_a.wait()
        cp_b.wait()
        if c + 1 < NCHUNK:
            inflight = start(c + 1)

        @plsc.parallel_loop(0, CHUNK, unroll=UNROLL)
        def _dot_body(r, c=c, arows=arows, brows=brows):
            acc0 = (arows[r, pl.ds(0, L)] * brows[r, pl.ds(0, L)])
            acc1 = (arows[r, pl.ds(L, L)] * brows[r, pl.ds(L, L)])
            for j in range(2, EMBED_SIZE // L, 2):
                acc0 = acc0 + arows[r, pl.ds(j * L, L)] * brows[r, pl.ds(j * L, L)]
                acc1 = acc1 + arows[r, pl.ds((j + 1) * L, L)] * brows[r, pl.ds((j + 1) * L, L)]
            cum = jnp.cumsum(acc0 + acc1)
            plsc.store_scatter(
                dots, [jnp.full((L,), c * CHUNK + r, jnp.int32)],
                cum, mask=mask_last)

    @plsc.parallel_loop(0, B_PER_W // L, unroll=UNROLL)
    def _loss_body(i):
        off = pl.multiple_of(i * L, L)
        t = sign_v[pl.ds(off, L)] * dots[pl.ds(off, L)]
        u = jnp.exp(-jnp.abs(t))
        loss = jnp.maximum(-t, 0.0) + _log1p_poly(u)
        dots[pl.ds(off, L)] = loss

    pltpu.sync_copy(dots, out_hbm.at[pl.ds(base, B_PER_W)])


@jax.jit
def _run(a2d, b2d, sign, emb_table, ctx_table):
    mesh = plsc.VectorSubcoreMesh(core_axis_name="c", subcore_axis_name="s")
    f = pl.kernel(
        _sc_kernel,
        mesh=mesh,
        compiler_params=pltpu.CompilerParams(needs_layout_passes=False),
        out_type=jax.ShapeDtypeStruct((BATCH,), jnp.float32),
        scratch_types=[
            pltpu.VMEM((NCHUNK, CHUNK), jnp.int32),
            pltpu.VMEM((NCHUNK, CHUNK), jnp.int32),
            pltpu.VMEM((B_PER_W,), jnp.float32),
            pltpu.VMEM((CHUNK, EMBED_SIZE), jnp.float32),
            pltpu.VMEM((CHUNK, EMBED_SIZE), jnp.float32),
            pltpu.VMEM((CHUNK, EMBED_SIZE), jnp.float32),
            pltpu.VMEM((CHUNK, EMBED_SIZE), jnp.float32),
            pltpu.VMEM((B_PER_W,), jnp.float32),
            pltpu.SemaphoreType.DMA,
            pltpu.SemaphoreType.DMA,
            pltpu.SemaphoreType.DMA,
            pltpu.SemaphoreType.DMA,
        ],
    )
    return f(a2d, b2d, sign, emb_table, ctx_table)


def kernel(a, b, sign, emb_table, ctx_table):
    a2d = a.reshape(BATCH // CHUNK, CHUNK)
    b2d = b.reshape(BATCH // CHUNK, CHUNK)
    return _run(a2d, b2d, sign, emb_table, ctx_table)
